# Initial kernel scaffold; baseline (speedup 1.0000x reference)
import jax
import jax.numpy as jnp
from jax.experimental import pallas as pl


def kernel(input_tensor, go_ke, ke_ke, tissue, W_dec, b_dec, W_gk, b_gk, W_self1, W_nbr1, b_kk1, W_self2, W_nbr2, b_kk2, W_ke, b_ke, W_bio1, b_bio1, W_bio2, b_bio2, W_dr1, b_dr1, W_dr2, b_dr2, W_pred, b_pred):
    raise NotImplementedError("write your pallas kernel here")


# speedup vs baseline: 18.4565x; 18.4565x over previous
import functools

import jax
import jax.numpy as jnp
import numpy as np
from jax import lax
from jax.experimental import pallas as pl
from jax.experimental.pallas import tpu as pltpu

from jax.experimental.pallas import tpu_sc as plsc

B = 32
N_GO = 16384
N_KE = 4096
D = 8
E_GK = 131072
E_KK = 65536
N_TISSUE = 512
DRUG = 2048
C = B * D

NC = 2
NS = 16
CH = 64


CL = C // NS


CHI = 2048


def _sc_edge_stage(table, src_idx, dst_idx, nch):
    mesh = plsc.VectorSubcoreMesh(core_axis_name="c", subcore_axis_name="s")

    @functools.partial(
        pl.kernel,
        out_type=jax.ShapeDtypeStruct((NC, NS, N_KE * CL), jnp.float32),
        mesh=mesh,
        scratch_types=[
            pltpu.VMEM((CHI,), jnp.int32),
            pltpu.VMEM((CHI,), jnp.int32),
            pltpu.VMEM((16, 16, CL), jnp.float32),
            pltpu.VMEM((N_KE * CL,), jnp.float32),
            pltpu.SemaphoreType.DMA,
        ],
    )
    def k(tab_hbm, src_hbm, dst_hbm, out_hbm, sidx, didx, slots, acc, sem):
        c = lax.axis_index("c")
        s = lax.axis_index("s")
        col0 = s * CL
        G = CHI // 16

        zv = jnp.zeros((CL,), jnp.float32)

        def zro(r, carry):
            for w in range(8):
                acc[pl.ds((r * 8 + w) * CL, CL)] = zv
            return carry
        lax.fori_loop(0, N_KE // 8, zro, 0)

        def issue(g, bank):
            base = g * 16
            offv = sidx[pl.ds(base, 16)] * C + col0
            for q in range(16):
                off = pl.multiple_of(offv[q], 16)
                pltpu.make_async_copy(
                    tab_hbm.at[pl.ds(off, CL)], slots.at[bank, q], sem).start()

        def drain_accum(g, bank):
            dvec = didx[pl.ds(g * 16, 16)] * CL
            offs = [dvec[q] for q in range(16)]
            for q in range(16):
                pltpu.make_async_copy(
                    tab_hbm.at[pl.ds(0, CL)], slots.at[bank, q], sem).wait()
            for q in range(16):
                plsc.addupdate(acc.at[pl.ds(offs[q], CL)],
                               slots[bank, q, :])

        def chunk(j, carry):
            pltpu.sync_copy(src_hbm.at[c, j], sidx)
            pltpu.sync_copy(dst_hbm.at[c, j], didx)
            for p in range(15):
                issue(p, p)

            def grp(g, cc):
                nxt = g + 15

                @pl.when(nxt < G)
                def _():
                    issue(nxt, nxt % 16)
                drain_accum(g, g % 16)
                return cc
            lax.fori_loop(0, G, grp, 0)
            return carry
        lax.fori_loop(0, nch, chunk, 0)

        pltpu.sync_copy(acc, out_hbm.at[c, s])

    return k(table.reshape(-1), src_idx, dst_idx)


def _table_body(xT_ref, R_ref, wdec_ref, bdec_ref, BDgk_ref, bgk_ref, out_ref):
    xe = jnp.dot(xT_ref[...], R_ref[...], preferred_element_type=jnp.float32, precision=lax.Precision.HIGHEST)
    h = jnp.tanh(xe * wdec_ref[...] + bdec_ref[...])
    out_ref[...] = jnp.tanh(
        jnp.dot(h, BDgk_ref[...], preferred_element_type=jnp.float32)
        + bgk_ref[...]
    )


def _build_table(xT, R, wdec_row, bdec_row, BDgk, bgk_row):
    GB = 2048
    grid = (N_GO // GB,)
    return pl.pallas_call(
        _table_body,
        grid=grid,
        in_specs=[
            pl.BlockSpec((GB, B), lambda i: (i, 0)),
            pl.BlockSpec((B, C), lambda i: (0, 0)),
            pl.BlockSpec((1, C), lambda i: (0, 0)),
            pl.BlockSpec((1, C), lambda i: (0, 0)),
            pl.BlockSpec((C, C), lambda i: (0, 0)),
            pl.BlockSpec((1, C), lambda i: (0, 0)),
        ],
        out_specs=pl.BlockSpec((GB, C), lambda i: (i, 0)),
        out_shape=jax.ShapeDtypeStruct((N_GO, C), jnp.float32),
        interpret=False,
    )(xT, R, wdec_row, bdec_row, BDgk, bgk_row)


def _unshard(p):
    return jnp.moveaxis(p.reshape(NC, NS, N_KE, CL), 1, 2).reshape(
        NC, N_KE, C)


def _psum_body(p0_ref, p1_ref, out_ref):
    out_ref[...] = p0_ref[...] + p1_ref[...]


def _partial_sum(p):
    pt = _unshard(p)
    return pl.pallas_call(
        _psum_body,
        out_shape=jax.ShapeDtypeStruct((N_KE, C), jnp.float32),
        interpret=False,
    )(pt[0], pt[1])


def _layer_body(ke_ref, p0_ref, p1_ref, BDs_ref, BDn_ref, b_ref, out_ref):
    agg = p0_ref[...] + p1_ref[...]
    out_ref[...] = jnp.tanh(
        jnp.dot(ke_ref[...], BDs_ref[...], preferred_element_type=jnp.float32)
        + jnp.dot(agg, BDn_ref[...], preferred_element_type=jnp.float32)
        + b_ref[...]
    )


def _layer_update(ke, p, BDs, BDn, b_row):
    pt = _unshard(p)
    return pl.pallas_call(
        _layer_body,
        out_shape=jax.ShapeDtypeStruct((N_KE, C), jnp.float32),
        interpret=False,
    )(ke, pt[0], pt[1], BDs, BDn, b_row)


def _final_body(ke_ref, Wke_ref, bke_ref, tis_ref, Wb1_ref, bb1_ref, Wb2_ref,
                bb2_ref, drug_ref, Wd1_ref, bd1_ref, Wd2_ref, bd2_ref,
                Wpb_ref, Wpd_ref, bp_ref, out_ref):
    ks = (
        jnp.dot(ke_ref[...], Wke_ref[...], preferred_element_type=jnp.float32)
        + bke_ref[0, 0]
    )
    onehot = (
        lax.broadcasted_iota(jnp.int32, (N_TISSUE, N_KE), 1) == tis_ref[...]
    ).astype(jnp.float32)
    bioT = jnp.dot(onehot, ks, preferred_element_type=jnp.float32, precision=lax.Precision.HIGHEST)
    bio = lax.dot_general(
        bioT, Wb1_ref[...], (((0,), (0,)), ((), ())),
        preferred_element_type=jnp.float32,
    ) + bb1_ref[...]
    bio = jax.nn.relu(bio)
    bio = jax.nn.relu(
        jnp.dot(bio, Wb2_ref[...], preferred_element_type=jnp.float32)
        + bb2_ref[...]
    )
    dr = jax.nn.relu(
        jnp.dot(drug_ref[...], Wd1_ref[...], preferred_element_type=jnp.float32)
        + bd1_ref[...]
    )
    dr = jax.nn.relu(
        jnp.dot(dr, Wd2_ref[...], preferred_element_type=jnp.float32)
        + bd2_ref[...]
    )
    out_ref[...] = (
        jnp.dot(bio, Wpb_ref[...], preferred_element_type=jnp.float32)
        + jnp.dot(dr, Wpd_ref[...], preferred_element_type=jnp.float32)
        + bp_ref[0, 0]
    )


def _final_stage(ke2, Wke_mat, bke, tissue2d, Wb1, bb1, Wb2, bb2, drug_in,
                 Wd1, bd1, Wd2, bd2, Wp_bio, Wp_drug, bpred):
    return pl.pallas_call(
        _final_body,
        out_shape=jax.ShapeDtypeStruct((B, 1), jnp.float32),
        interpret=False,
    )(ke2, Wke_mat, bke, tissue2d, Wb1, bb1, Wb2, bb2, drug_in,
      Wd1, bd1, Wd2, bd2, Wp_bio, Wp_drug, bpred)


def kernel(input_tensor, go_ke, ke_ke, tissue, W_dec, b_dec, W_gk, b_gk,
           W_self1, W_nbr1, b_kk1, W_self2, W_nbr2, b_kk2, W_ke, b_ke,
           W_bio1, b_bio1, W_bio2, b_bio2, W_dr1, b_dr1, W_dr2, b_dr2,
           W_pred, b_pred):
    f32 = jnp.float32
    eyeB = jnp.eye(B, dtype=f32)
    R = jnp.kron(eyeB, jnp.ones((1, D), f32))
    wdec_row = jnp.tile(W_dec[0], B)[None, :]
    bdec_row = jnp.tile(b_dec, B)[None, :]
    BDgk = jnp.kron(eyeB, W_gk)
    bgk_row = jnp.tile(b_gk, B)[None, :]
    BDs1 = jnp.kron(eyeB, W_self1)
    BDn1 = jnp.kron(eyeB, W_nbr1)
    b1_row = jnp.tile(b_kk1, B)[None, :]
    BDs2 = jnp.kron(eyeB, W_self2)
    BDn2 = jnp.kron(eyeB, W_nbr2)
    b2_row = jnp.tile(b_kk2, B)[None, :]
    Wke_mat = jnp.kron(eyeB, W_ke)

    xT = input_tensor[:, :N_GO].T
    drug_in = input_tensor[:, N_GO:]

    nch_gk = E_GK // (NC * CHI)
    nch_kk = E_KK // (NC * CHI)
    src_gk = go_ke[0].reshape(NC, nch_gk, CHI)
    dst_gk = go_ke[1].reshape(NC, nch_gk, CHI)
    src_kk = ke_ke[0].reshape(NC, nch_kk, CHI)
    dst_kk = ke_ke[1].reshape(NC, nch_kk, CHI)

    T_all = _build_table(xT, R, wdec_row, bdec_row, BDgk, bgk_row)

    ke = _partial_sum(
        _sc_edge_stage(T_all, src_gk, dst_gk, nch_gk))

    pa1 = _sc_edge_stage(ke, src_kk, dst_kk, nch_kk)
    ke = _layer_update(ke, pa1, BDs1, BDn1, b1_row)
    pa2 = _sc_edge_stage(ke, src_kk, dst_kk, nch_kk)
    ke = _layer_update(ke, pa2, BDs2, BDn2, b2_row)

    return _final_stage(
        ke, Wke_mat, b_ke.reshape(1, 1), tissue.reshape(N_TISSUE, 1),
        W_bio1, b_bio1[None, :], W_bio2, b_bio2[None, :], drug_in,
        W_dr1, b_dr1[None, :], W_dr2, b_dr2[None, :],
        W_pred[:128], W_pred[128:], b_pred.reshape(1, 1),
    )
